# baseline (speedup 1.0000x reference)
import functools

import jax
import jax.numpy as jnp
from jax import lax
from jax.experimental import pallas as pl
from jax.experimental.pallas import tpu as pltpu
from jax.experimental.pallas import tpu_sc as plsc

_BLK = 128
_STRIDE = 256
_CTX = 8


def _sc_gather_body(elems_per, blocks_per_seq, lut_hbm, ids_hbm, aux_hbm,
                    out_hbm, idx_v, val_v, pad_v, sem):
    nc = plsc.get_sparse_core_info().num_cores
    wid = (lax.axis_index("s") * jnp.int32(nc)
           + lax.axis_index("c")).astype(jnp.int32)
    base = wid * jnp.int32(elems_per)
    n_blocks = elems_per // _BLK
    chunks_per_seq = blocks_per_seq // n_blocks
    seq_start = (wid % jnp.int32(chunks_per_seq)) == 0

    pltpu.sync_copy(aux_hbm.at[pl.ds(0, 2 * _CTX)], pad_v)
    pltpu.sync_copy(ids_hbm.at[pl.ds(base, elems_per)],
                    idx_v.at[pl.ds(_CTX * 2, elems_per)])

    @pl.when(wid > 0)
    def _():
        pltpu.sync_copy(ids_hbm.at[pl.ds(base - _CTX, _CTX)],
                        idx_v.at[pl.ds(_CTX, _CTX)])

    @pl.when(wid == 0)
    def _():
        idx_v[pl.ds(0, 2 * _CTX)] = jnp.zeros((2 * _CTX,), jnp.int32)

    copies = [
        pltpu.async_copy(
            lut_hbm.at[idx_v.at[pl.ds(_CTX * 2 + j * _BLK, _BLK)]],
            val_v.at[pl.ds(_CTX * 2 + j * _BLK, _BLK)], sem)
        for j in range(n_blocks)
    ]
    copies.append(
        pltpu.async_copy(lut_hbm.at[idx_v.at[pl.ds(_CTX, _CTX)]],
                         val_v.at[pl.ds(_CTX, _CTX)], sem))
    for c in copies:
        c.wait()

    for j in range(n_blocks):
        r = wid * jnp.int32(n_blocks) + jnp.int32(j)
        out0 = r * jnp.int32(_STRIDE)
        pltpu.sync_copy(val_v.at[pl.ds(_CTX * 2 + j * _BLK, _BLK)],
                        out_hbm.at[pl.ds(out0 + jnp.int32(_CTX), _BLK)])
        if j == 0:
            @pl.when(seq_start)
            def _():
                pltpu.sync_copy(pad_v.at[pl.ds(0, _CTX)],
                                out_hbm.at[pl.ds(out0, _CTX)])

            @pl.when(jnp.logical_not(seq_start))
            def _():
                pltpu.sync_copy(val_v.at[pl.ds(_CTX, _CTX)],
                                out_hbm.at[pl.ds(out0, _CTX)])
        else:
            pltpu.sync_copy(val_v.at[pl.ds(_CTX * 2 + j * _BLK - _CTX, _CTX)],
                            out_hbm.at[pl.ds(out0, _CTX)])


def _sc_gather(lut32, ids_flat, aux32, blocks_per_seq):
    n = ids_flat.shape[0]
    info = plsc.get_sparse_core_info()
    num_workers = info.num_cores * info.num_subcores
    elems_per = n // num_workers
    mesh = plsc.VectorSubcoreMesh(core_axis_name="c", subcore_axis_name="s")

    body = functools.partial(_sc_gather_body, elems_per, blocks_per_seq)
    return pl.kernel(
        body,
        out_type=jax.ShapeDtypeStruct((n // _BLK * _STRIDE,), jnp.int32),
        mesh=mesh,
        scratch_types=[
            pltpu.VMEM((2 * _CTX + elems_per,), jnp.int32),
            pltpu.VMEM((2 * _CTX + elems_per,), jnp.int32),
            pltpu.VMEM((2 * _CTX,), jnp.int32),
            pltpu.SemaphoreType.DMA,
        ],
    )(lut32, ids_flat, aux32)


def _hash_body(max_ngram, n_head, n_rows, aux_ref, s_ref, out_ref):
    mask16 = jnp.uint32(0xFFFF)
    view = s_ref[...].reshape(n_rows, _STRIDE)

    prods = []
    for k in range(max_ngram):
        s = view[:, _CTX - k:_CTX - k + _BLK].astype(jnp.uint32)
        s0 = s & mask16
        s1_nz = (s >> 16) > 0
        mk0 = aux_ref[2 * _CTX + 3 * k].astype(jnp.uint32)
        mk1 = aux_ref[2 * _CTX + 3 * k + 1].astype(jnp.uint32)
        mk2 = aux_ref[2 * _CTX + 3 * k + 2].astype(jnp.uint32)
        a0 = s0 * mk0
        a1 = s0 * mk1
        a2 = s0 * mk2
        b0 = jnp.where(s1_nz, mk0, jnp.uint32(0))
        b1 = jnp.where(s1_nz, mk1, jnp.uint32(0))
        b2 = jnp.where(s1_nz, mk2, jnp.uint32(0))
        c0 = a0 & mask16
        t1 = (a0 >> 16) + (a1 & mask16) + b0
        t2 = (t1 >> 16) + (a1 >> 16) + (a2 & mask16) + b1
        t3 = (t2 >> 16) + (a2 >> 16) + b2
        lo = c0 | ((t1 & mask16) << 16)
        hi = (t2 & mask16) | ((t3 & mask16) << 16)
        prods.append((hi, lo))

    mix_hi, mix_lo = prods[0]
    idx = 0
    for n in range(2, max_ngram + 1):
        mix_hi = mix_hi ^ prods[n - 1][0]
        mix_lo = mix_lo ^ prods[n - 1][1]
        hi_s = mix_hi.astype(jnp.int32)
        l1 = (mix_lo >> 16).astype(jnp.int32)
        l0 = (mix_lo & mask16).astype(jnp.int32)
        for _ in range(n_head):
            p = aux_ref[2 * _CTX + 3 * max_ngram + idx]
            inv = jnp.float32(1.0) / p.astype(jnp.float32)

            def fmod31(y):
                q = (y.astype(jnp.float32) * inv).astype(jnp.int32)
                r = y - q * p
                r = jnp.where(r < 0, r + p, r)
                return jnp.where(r >= p, r - p, r)

            acc = fmod31(hi_s)
            acc = fmod31(acc << 14)
            acc = fmod31(((acc << 2) + l1) << 12)
            acc = fmod31((acc << 4) + l0)
            out_ref[idx] = acc
            idx += 1


def _tc_hash(max_ngram, n_head, aux32, s_ext, interpret=False):
    n_rows = s_ext.shape[0] // _STRIDE
    n_out = (max_ngram - 1) * n_head
    return pl.pallas_call(
        functools.partial(_hash_body, max_ngram, n_head, n_rows),
        out_shape=jax.ShapeDtypeStruct((n_out, n_rows, _BLK), jnp.int32),
        in_specs=[
            pl.BlockSpec(memory_space=pltpu.SMEM),
            pl.BlockSpec(memory_space=pltpu.VMEM),
        ],
        out_specs=pl.BlockSpec(memory_space=pltpu.VMEM),
        interpret=interpret,
    )(aux32, s_ext)


def _build_aux(multipliers, prime_mods, pad_id, max_ngram):
    pad64 = jnp.asarray(pad_id).astype(jnp.int64)
    limbs = jnp.stack(
        [(multipliers[k] >> (16 * j)) & 0xFFFF
         for k in range(max_ngram) for j in range(3)])
    return jnp.concatenate(
        [jnp.broadcast_to(pad64[None], (2 * _CTX,)), limbs, prime_mods]
    ).astype(jnp.int32)


def kernel(input_ids, lookup_table, multipliers, prime_mods, pad_id):
    b, t = input_ids.shape
    max_ngram = multipliers.shape[0]
    n_head = prime_mods.shape[0] // (max_ngram - 1)

    vmax = lookup_table.shape[0] - 1
    ids_flat = jnp.clip(input_ids.reshape(-1), 0, vmax).astype(jnp.int32)
    lut32 = lookup_table.astype(jnp.int32)
    aux32 = _build_aux(multipliers, prime_mods, pad_id, max_ngram)

    s_ext = _sc_gather(lut32, ids_flat, aux32, t // _BLK)
    out = _tc_hash(max_ngram, n_head, aux32, s_ext)
    out = out.reshape(prime_mods.shape[0], b, t)
    return jnp.transpose(out, (1, 2, 0)).astype(jnp.int64)
